# Initial kernel scaffold; baseline (speedup 1.0000x reference)
import jax
import jax.numpy as jnp
from jax.experimental import pallas as pl


def kernel(x, tables):
    raise NotImplementedError("write your pallas kernel here")


# speedup vs baseline: 9.1441x; 9.1441x over previous
import functools
import jax
import jax.numpy as jnp
from jax import lax
from jax.experimental import pallas as pl
from jax.experimental.pallas import tpu as pltpu
from jax.experimental.pallas import tpu_sc as plsc

NUM_CHANNELS = 8
VOCAB_P1 = 1001
EMBED = 128

NC, NS, L = 2, 16, 16
NW = NC * NS

CH = 128


def _body(tab_hbm, idx_hbm, out_hbm, idx_a, idx_b, rows_a, rows_b, sem_a, sem_b):
    wid = lax.axis_index("s") * NC + lax.axis_index("c")
    b_per_w = idx_hbm.shape[0] // NW
    n_ch = b_per_w // CH
    offs = lax.rem(lax.iota(jnp.int32, L), NUM_CHANNELS) * VOCAB_P1

    def stage(g, idx_v, rows_v, sem):
        base = pl.multiple_of(wid * b_per_w + g * CH, CH)
        pltpu.sync_copy(idx_hbm.at[pl.ds(base, CH)], idx_v)
        for j in range(CH // L):
            idx_v[pl.ds(j * L, L)] = idx_v[pl.ds(j * L, L)] + offs
        return pltpu.async_copy(tab_hbm.at[idx_v], rows_v, sem)

    def drain(g, rows_v, copy):
        copy.wait()
        base = pl.multiple_of(wid * b_per_w + g * CH, CH)
        pltpu.sync_copy(rows_v, out_hbm.at[pl.ds(base, CH)])

    cp0 = stage(0, idx_a, rows_a, sem_a)

    def outer(gg, carry):
        g = gg * 2
        cp1 = stage(g + 1, idx_b, rows_b, sem_b)
        drain(g, rows_a, cp0)
        cp0b = stage(g + 2, idx_a, rows_a, sem_a)
        drain(g + 1, rows_b, cp1)
        return carry

    lax.fori_loop(0, n_ch // 2 - 1, outer, 0)
    g = n_ch - 2
    cp1 = stage(g + 1, idx_b, rows_b, sem_b)
    drain(g, rows_a, cp0)
    drain(g + 1, rows_b, cp1)


def kernel(x, tables):
    B, T, C = x.shape
    flat_idx = x.reshape(-1).astype(jnp.int32)
    flat_tab = tables.reshape(-1, EMBED)
    n = flat_idx.shape[0]

    k = pl.kernel(
        _body,
        out_type=jax.ShapeDtypeStruct((n, EMBED), jnp.float32),
        mesh=plsc.VectorSubcoreMesh(core_axis_name="c", subcore_axis_name="s"),
        scratch_types=[
            pltpu.VMEM((CH,), jnp.int32),
            pltpu.VMEM((CH,), jnp.int32),
            pltpu.VMEM((CH, EMBED), jnp.float32),
            pltpu.VMEM((CH, EMBED), jnp.float32),
            pltpu.SemaphoreType.DMA,
            pltpu.SemaphoreType.DMA,
        ],
    )
    out = k(flat_tab, flat_idx)
    return out.reshape(B, T, C, EMBED)


# speedup vs baseline: 9.8770x; 1.0801x over previous
import jax
import jax.numpy as jnp
from jax import lax
from jax.experimental import pallas as pl
from jax.experimental.pallas import tpu as pltpu
from jax.experimental.pallas import tpu_sc as plsc

NUM_CHANNELS = 8
VOCAB_P1 = 1001
EMBED = 128

NC, NS, L = 2, 16, 16
NW = NC * NS

CH = 128
NBUF = 4
W = 2


def _body(tab_hbm, idx_hbm, out_hbm, idx_all,
          r0, r1, r2, r3, gs0, gs1, gs2, gs3, ws0, ws1, ws2, ws3):
    rows = [r0, r1, r2, r3]
    gsem = [gs0, gs1, gs2, gs3]
    wsem = [ws0, ws1, ws2, ws3]
    n_ch = idx_all.shape[0]
    wid = lax.axis_index("s") * NC + lax.axis_index("c")
    row_base = wid * n_ch

    pltpu.sync_copy(idx_hbm.at[wid], idx_all)
    offs = lax.rem(lax.iota(jnp.int32, L), NUM_CHANNELS) * VOCAB_P1

    def add_off(t, c):
        r = idx_all.at[t]
        for j in range(CH // L):
            r[pl.ds(j * L, L)] = r[pl.ds(j * L, L)] + offs
        return c

    lax.fori_loop(0, n_ch, add_off, 0)

    def g_start(g, b):
        pltpu.make_async_copy(tab_hbm.at[idx_all.at[g]], rows[b], gsem[b]).start()

    def g_wait(g, b):
        pltpu.make_async_copy(tab_hbm.at[idx_all.at[g]], rows[b], gsem[b]).wait()

    def w_start(g, b):
        base = pl.multiple_of((row_base + g) * CH, CH)
        pltpu.make_async_copy(rows[b], out_hbm.at[pl.ds(base, CH)], wsem[b]).start()

    def w_wait(g, b):
        base = pl.multiple_of((row_base + g) * CH, CH)
        pltpu.make_async_copy(rows[b], out_hbm.at[pl.ds(base, CH)], wsem[b]).wait()

    g_start(0, 0)
    g_start(1, 1)
    g_start(2, 2)
    g_wait(0, 0)
    w_start(0, 0)
    g_start(3, 3)
    g_wait(1, 1)
    w_start(1, 1)

    def outer(q, c):
        g0 = q * NBUF
        for b in range(NBUF):
            g = g0 + b
            w_wait(g - NBUF, b)
            g_start(g, b)
            b2 = (b - W) % NBUF
            g_wait(g - W, b2)
            w_start(g - W, b2)
        return c

    lax.fori_loop(1, n_ch // NBUF, outer, 0)

    g_wait(n_ch - 2, (NBUF - 2) % NBUF)
    w_start(n_ch - 2, (NBUF - 2) % NBUF)
    g_wait(n_ch - 1, NBUF - 1)
    w_start(n_ch - 1, NBUF - 1)
    for b in range(NBUF):
        w_wait(n_ch - NBUF + b, b)


def kernel(x, tables):
    B, T, C = x.shape
    n = B * T * C
    n_ch = n // CH // NW
    flat_idx = x.reshape(NW, n_ch, CH).astype(jnp.int32)
    flat_tab = tables.reshape(-1, EMBED)

    k = pl.kernel(
        _body,
        out_type=jax.ShapeDtypeStruct((n, EMBED), jnp.float32),
        mesh=plsc.VectorSubcoreMesh(core_axis_name="c", subcore_axis_name="s"),
        scratch_types=(
            [pltpu.VMEM((n_ch, CH), jnp.int32)]
            + [pltpu.VMEM((CH, EMBED), jnp.float32)] * NBUF
            + [pltpu.SemaphoreType.DMA] * (2 * NBUF)
        ),
    )
    out = k(flat_tab, flat_idx)
    return out.reshape(B, T, C, EMBED)
